# Initial kernel scaffold; baseline (speedup 1.0000x reference)
import jax
import jax.numpy as jnp
from jax.experimental import pallas as pl


def kernel(feats, edge_index, max_nodes, W0, b0, W1, b1):
    raise NotImplementedError("write your pallas kernel here")


# speedup vs baseline: 3.7491x; 3.7491x over previous
import functools

import jax
import jax.numpy as jnp
from jax import lax
from jax.experimental import pallas as pl
from jax.experimental.pallas import tpu as pltpu
from jax.experimental.pallas import tpu_sc as plsc

_NC = 2
_NS = 16
_NW = _NC * _NS
_K = 80
_C0 = 104


@functools.lru_cache(maxsize=None)
def _make_sc_agg(N, D, E):
    e_per_w = E // _NW
    assert e_per_w * _NW == E and e_per_w % _K == 0
    nchunk = e_per_w // _K
    stripe = (N // _NS) // 8 * 8
    rem = N - stripe * _NS
    assert rem % 8 == 0 and 0 <= rem <= _C0
    assert stripe % _C0 == 0

    mesh = plsc.VectorSubcoreMesh(core_axis_name="c", subcore_axis_name="s")

    out_type = jax.ShapeDtypeStruct((_NC, N, D), jnp.float32)
    scratch = [
        pltpu.VMEM((_K,), jnp.int32),
        pltpu.VMEM((_K,), jnp.int32),
        pltpu.VMEM((_K, D), jnp.float32),
        pltpu.VMEM((_C0, D), jnp.float32),
        pltpu.VMEM_SHARED((N, D), jnp.float32),
        pltpu.SemaphoreType.DMA,
    ]

    def body(zrows_hbm, h_hbm, src_hbm, dst_hbm, agg_out,
             src_v, dst_v, rows_v, stage_v, agg_sh, sem):
        c = lax.axis_index("c")
        s = lax.axis_index("s")
        wid = s * _NC + c
        r0 = s * stripe
        last = s == (_NS - 1)

        pltpu.sync_copy(zrows_hbm, stage_v)
        for j in range(stripe // _C0):
            pltpu.sync_copy(stage_v, agg_sh.at[pl.ds(r0 + j * _C0, _C0)])
        if rem:
            @pl.when(last)
            def _():
                pltpu.sync_copy(stage_v.at[pl.ds(0, rem)],
                                agg_sh.at[pl.ds(_NS * stripe, rem)])
        plsc.subcore_barrier()

        base0 = wid * e_per_w

        def step(i, carry):
            base = base0 + i * _K
            pltpu.sync_copy(src_hbm.at[pl.ds(base, _K)], src_v)
            pltpu.sync_copy(dst_hbm.at[pl.ds(base, _K)], dst_v)
            pltpu.async_copy(h_hbm.at[src_v], rows_v, sem).wait()
            pltpu.sync_copy(rows_v, agg_sh.at[dst_v], add=True)
            return carry

        lax.fori_loop(0, nchunk, step, 0)
        plsc.subcore_barrier()

        def publish(nrows, base):
            pltpu.sync_copy(agg_sh.at[pl.ds(base, nrows)],
                            stage_v.at[pl.ds(0, nrows)])
            pltpu.sync_copy(stage_v.at[pl.ds(0, nrows)],
                            agg_out.at[c, pl.ds(base, nrows)])

        for j in range(stripe // _C0):
            publish(_C0, r0 + j * _C0)
        if rem:
            @pl.when(last)
            def _():
                publish(rem, _NS * stripe)

    return pl.kernel(body, out_type=out_type, mesh=mesh, scratch_types=scratch)


def _dense_body(h_ref, agg_a, agg_b, deg_a, deg_b, w_top, w_bot, b_ref,
                out_ref, *, softmax):
    deg = deg_a[:, :1] + deg_b[:, :1]
    mean = (agg_a[...] + agg_b[...]) / jnp.maximum(deg, 1.0)
    z = (jnp.dot(h_ref[...], w_top[...], preferred_element_type=jnp.float32)
         + jnp.dot(mean, w_bot[...], preferred_element_type=jnp.float32)
         + b_ref[...])
    z = jnp.maximum(z, 0.0)
    if softmax:
        z = z - jnp.max(z, axis=-1, keepdims=True)
        e = jnp.exp(z)
        z = e / jnp.sum(e, axis=-1, keepdims=True)
    out_ref[...] = z


@functools.lru_cache(maxsize=None)
def _make_dense(N, D, F, softmax):
    BN = 400
    assert N % BN == 0
    grid = (N // BN,)
    return pl.pallas_call(
        functools.partial(_dense_body, softmax=softmax),
        grid=grid,
        in_specs=[
            pl.BlockSpec((BN, D), lambda i: (i, 0)),
            pl.BlockSpec((BN, D), lambda i: (i, 0)),
            pl.BlockSpec((BN, D), lambda i: (i, 0)),
            pl.BlockSpec((BN, 128), lambda i: (i, 0)),
            pl.BlockSpec((BN, 128), lambda i: (i, 0)),
            pl.BlockSpec((D, F), lambda i: (0, 0)),
            pl.BlockSpec((D, F), lambda i: (0, 0)),
            pl.BlockSpec((1, F), lambda i: (0, 0)),
        ],
        out_specs=pl.BlockSpec((BN, F), lambda i: (i, 0)),
        out_shape=jax.ShapeDtypeStruct((N, F), jnp.float32),
    )


def kernel(feats, edge_index, max_nodes, W0, b0, W1, b1):
    del max_nodes
    N, D = feats.shape
    E = edge_index.shape[1]
    F0 = W0.shape[1]
    F1 = W1.shape[1]
    src = edge_index[0]
    dst = edge_index[1]
    zrows = jnp.zeros((_C0, D), jnp.float32)
    ones_table = jnp.ones((N, D), jnp.float32)

    sc_agg = _make_sc_agg(N, D, E)

    deg = sc_agg(zrows, ones_table, src, dst)
    agg0 = sc_agg(zrows, feats, src, dst)
    h1 = _make_dense(N, D, F0, False)(
        feats, agg0[0], agg0[1], deg[0], deg[1], W0[:D], W0[D:],
        b0.reshape(1, F0))
    agg1 = sc_agg(zrows, h1, src, dst)
    out = _make_dense(N, F0, F1, True)(
        h1, agg1[0], agg1[1], deg[0], deg[1], W1[:F0], W1[F0:],
        b1.reshape(1, F1))
    return out


# speedup vs baseline: 4.5821x; 1.2222x over previous
import functools

import jax
import jax.numpy as jnp
from jax import lax
from jax.experimental import pallas as pl
from jax.experimental.pallas import tpu as pltpu
from jax.experimental.pallas import tpu_sc as plsc

_NC = 2
_NS = 16
_NW = _NC * _NS
_K = 80
_C0 = 104


@functools.lru_cache(maxsize=None)
def _make_sc_agg(N, D, E, gather=True):
    e_per_w = E // _NW
    assert e_per_w * _NW == E and e_per_w % _K == 0
    nchunk = e_per_w // _K
    stripe = (N // _NS) // 8 * 8
    rem = N - stripe * _NS
    assert rem % 8 == 0 and 0 <= rem <= _C0
    assert stripe % _C0 == 0

    mesh = plsc.VectorSubcoreMesh(core_axis_name="c", subcore_axis_name="s")

    out_type = jax.ShapeDtypeStruct((_NC, N, D), jnp.float32)
    scratch = [
        pltpu.VMEM((_K,), jnp.int32),
        pltpu.VMEM((_K,), jnp.int32),
        pltpu.VMEM((_K, D), jnp.float32),
        pltpu.VMEM((_C0, D), jnp.float32),
        pltpu.VMEM_SHARED((N, D), jnp.float32),
        pltpu.SemaphoreType.DMA,
    ]

    def body(zrows_hbm, h_hbm, src_hbm, dst_hbm, agg_out,
             src_v, dst_v, rows_v, stage_v, agg_sh, sem):
        c = lax.axis_index("c")
        s = lax.axis_index("s")
        wid = s * _NC + c
        r0 = s * stripe
        last = s == (_NS - 1)

        pltpu.sync_copy(zrows_hbm, stage_v)
        for j in range(stripe // _C0):
            pltpu.sync_copy(stage_v, agg_sh.at[pl.ds(r0 + j * _C0, _C0)])
        if rem:
            @pl.when(last)
            def _():
                pltpu.sync_copy(stage_v.at[pl.ds(0, rem)],
                                agg_sh.at[pl.ds(_NS * stripe, rem)])
        plsc.subcore_barrier()

        base0 = wid * e_per_w
        if not gather:
            pltpu.sync_copy(h_hbm, rows_v)

        def step(i, carry):
            base = base0 + i * _K
            pltpu.sync_copy(dst_hbm.at[pl.ds(base, _K)], dst_v)
            if gather:
                pltpu.sync_copy(src_hbm.at[pl.ds(base, _K)], src_v)
                pltpu.async_copy(h_hbm.at[src_v], rows_v, sem).wait()
            pltpu.sync_copy(rows_v, agg_sh.at[dst_v], add=True)
            return carry

        lax.fori_loop(0, nchunk, step, 0)
        plsc.subcore_barrier()

        def publish(nrows, base):
            pltpu.sync_copy(agg_sh.at[pl.ds(base, nrows)],
                            stage_v.at[pl.ds(0, nrows)])
            pltpu.sync_copy(stage_v.at[pl.ds(0, nrows)],
                            agg_out.at[c, pl.ds(base, nrows)])

        for j in range(stripe // _C0):
            publish(_C0, r0 + j * _C0)
        if rem:
            @pl.when(last)
            def _():
                publish(rem, _NS * stripe)

    return pl.kernel(body, out_type=out_type, mesh=mesh, scratch_types=scratch)


def _dense_body(h_ref, agg_a, agg_b, deg_a, deg_b, w_top, w_bot, b_ref,
                out_ref, *, softmax):
    deg = deg_a[:, :1] + deg_b[:, :1]
    mean = (agg_a[...] + agg_b[...]) / jnp.maximum(deg, 1.0)
    z = (jnp.dot(h_ref[...], w_top[...], preferred_element_type=jnp.float32)
         + jnp.dot(mean, w_bot[...], preferred_element_type=jnp.float32)
         + b_ref[...])
    z = jnp.maximum(z, 0.0)
    if softmax:
        z = z - jnp.max(z, axis=-1, keepdims=True)
        e = jnp.exp(z)
        z = e / jnp.sum(e, axis=-1, keepdims=True)
    out_ref[...] = z


@functools.lru_cache(maxsize=None)
def _make_dense(N, D, F, softmax):
    BN = 400
    assert N % BN == 0
    grid = (N // BN,)
    return pl.pallas_call(
        functools.partial(_dense_body, softmax=softmax),
        grid=grid,
        in_specs=[
            pl.BlockSpec((BN, D), lambda i: (i, 0)),
            pl.BlockSpec((BN, D), lambda i: (i, 0)),
            pl.BlockSpec((BN, D), lambda i: (i, 0)),
            pl.BlockSpec((BN, 128), lambda i: (i, 0)),
            pl.BlockSpec((BN, 128), lambda i: (i, 0)),
            pl.BlockSpec((D, F), lambda i: (0, 0)),
            pl.BlockSpec((D, F), lambda i: (0, 0)),
            pl.BlockSpec((1, F), lambda i: (0, 0)),
        ],
        out_specs=pl.BlockSpec((BN, F), lambda i: (i, 0)),
        out_shape=jax.ShapeDtypeStruct((N, F), jnp.float32),
    )


def kernel(feats, edge_index, max_nodes, W0, b0, W1, b1):
    del max_nodes
    N, D = feats.shape
    E = edge_index.shape[1]
    F0 = W0.shape[1]
    F1 = W1.shape[1]
    src = edge_index[0]
    dst = edge_index[1]
    zrows = jnp.zeros((_C0, D), jnp.float32)
    ones_rows = jnp.ones((_K, D), jnp.float32)

    sc_agg = _make_sc_agg(N, D, E)

    deg = _make_sc_agg(N, D, E, False)(zrows, ones_rows, src, dst)
    agg0 = sc_agg(zrows, feats, src, dst)
    h1 = _make_dense(N, D, F0, False)(
        feats, agg0[0], agg0[1], deg[0], deg[1], W0[:D], W0[D:],
        b0.reshape(1, F0))
    agg1 = sc_agg(zrows, h1, src, dst)
    out = _make_dense(N, F0, F1, True)(
        h1, agg1[0], agg1[1], deg[0], deg[1], W1[:F0], W1[F0:],
        b1.reshape(1, F1))
    return out


# speedup vs baseline: 6.5506x; 1.4296x over previous
import functools

import jax
import jax.numpy as jnp
from jax import lax
from jax.experimental import pallas as pl
from jax.experimental.pallas import tpu as pltpu
from jax.experimental.pallas import tpu_sc as plsc

_NC = 2
_NS = 16
_NW = _NC * _NS
_K = 80
_C0 = 104


@functools.lru_cache(maxsize=None)
def _make_sc_agg(N, D, E, gather=True):
    e_per_w = E // _NW
    assert e_per_w * _NW == E and e_per_w % _K == 0
    nchunk = e_per_w // _K
    stripe = (N // _NS) // 8 * 8
    rem = N - stripe * _NS
    assert rem % 8 == 0 and 0 <= rem <= _C0
    assert stripe % _C0 == 0

    mesh = plsc.VectorSubcoreMesh(core_axis_name="c", subcore_axis_name="s")

    if gather:
        assert nchunk % 2 == 1 and nchunk >= 3
    out_type = jax.ShapeDtypeStruct((_NC, N, D), jnp.float32)
    scratch = [
        pltpu.VMEM((_K,), jnp.int32),
        pltpu.VMEM((_K,), jnp.int32),
        pltpu.VMEM((_K, D), jnp.float32),
        pltpu.VMEM((_K,), jnp.int32),
        pltpu.VMEM((_K,), jnp.int32),
        pltpu.VMEM((_K, D), jnp.float32),
        pltpu.VMEM((_C0, D), jnp.float32),
        pltpu.VMEM_SHARED((N, D), jnp.float32),
        pltpu.SemaphoreType.DMA,
        pltpu.SemaphoreType.DMA,
    ]

    def body(zrows_hbm, h_hbm, src_hbm, dst_hbm, agg_out,
             src_v, dst_v, rows_v, src_w, dst_w, rows_w, stage_v, agg_sh,
             sem0, sem1):
        c = lax.axis_index("c")
        s = lax.axis_index("s")
        wid = s * _NC + c
        r0 = s * stripe
        last = s == (_NS - 1)

        pltpu.sync_copy(zrows_hbm, stage_v)
        for j in range(stripe // _C0):
            pltpu.sync_copy(stage_v, agg_sh.at[pl.ds(r0 + j * _C0, _C0)])
        if rem:
            @pl.when(last)
            def _():
                pltpu.sync_copy(stage_v.at[pl.ds(0, rem)],
                                agg_sh.at[pl.ds(_NS * stripe, rem)])
        plsc.subcore_barrier()

        base0 = wid * e_per_w

        def start_gather(i, sv, dv, rv, sem):
            base = base0 + i * _K
            pltpu.sync_copy(src_hbm.at[pl.ds(base, _K)], sv)
            pltpu.sync_copy(dst_hbm.at[pl.ds(base, _K)], dv)
            pltpu.async_copy(h_hbm.at[sv], rv, sem)

        def drain(rv, sem):
            pltpu.make_async_copy(h_hbm.at[pl.ds(0, _K)], rv, sem).wait()

        if gather:
            start_gather(0, src_v, dst_v, rows_v, sem0)

            def step(j, carry):
                i0 = 2 * j
                start_gather(i0 + 1, src_w, dst_w, rows_w, sem1)
                drain(rows_v, sem0)
                pltpu.sync_copy(rows_v, agg_sh.at[dst_v], add=True)
                start_gather(i0 + 2, src_v, dst_v, rows_v, sem0)
                drain(rows_w, sem1)
                pltpu.sync_copy(rows_w, agg_sh.at[dst_w], add=True)
                return carry

            lax.fori_loop(0, (nchunk - 1) // 2, step, 0)
            drain(rows_v, sem0)
            pltpu.sync_copy(rows_v, agg_sh.at[dst_v], add=True)
        else:
            pltpu.sync_copy(h_hbm, rows_v)

            def step(i, carry):
                base = base0 + i * _K
                pltpu.sync_copy(dst_hbm.at[pl.ds(base, _K)], dst_v)
                pltpu.sync_copy(rows_v, agg_sh.at[dst_v], add=True)
                return carry

            lax.fori_loop(0, nchunk, step, 0)
        plsc.subcore_barrier()

        def publish(nrows, base):
            pltpu.sync_copy(agg_sh.at[pl.ds(base, nrows)],
                            stage_v.at[pl.ds(0, nrows)])
            pltpu.sync_copy(stage_v.at[pl.ds(0, nrows)],
                            agg_out.at[c, pl.ds(base, nrows)])

        for j in range(stripe // _C0):
            publish(_C0, r0 + j * _C0)
        if rem:
            @pl.when(last)
            def _():
                publish(rem, _NS * stripe)

    return pl.kernel(body, out_type=out_type, mesh=mesh, scratch_types=scratch)


def _dense_body(h_ref, agg_a, agg_b, deg_a, deg_b, w_top, w_bot, b_ref,
                out_ref, *, softmax):
    deg = deg_a[:, :1] + deg_b[:, :1]
    mean = (agg_a[...] + agg_b[...]) / jnp.maximum(deg, 1.0)
    z = (jnp.dot(h_ref[...], w_top[...], preferred_element_type=jnp.float32)
         + jnp.dot(mean, w_bot[...], preferred_element_type=jnp.float32)
         + b_ref[...])
    z = jnp.maximum(z, 0.0)
    if softmax:
        z = z - jnp.max(z, axis=-1, keepdims=True)
        e = jnp.exp(z)
        z = e / jnp.sum(e, axis=-1, keepdims=True)
    out_ref[...] = z


@functools.lru_cache(maxsize=None)
def _make_dense(N, D, F, softmax):
    BN = 400
    assert N % BN == 0
    grid = (N // BN,)
    return pl.pallas_call(
        functools.partial(_dense_body, softmax=softmax),
        grid=grid,
        in_specs=[
            pl.BlockSpec((BN, D), lambda i: (i, 0)),
            pl.BlockSpec((BN, D), lambda i: (i, 0)),
            pl.BlockSpec((BN, D), lambda i: (i, 0)),
            pl.BlockSpec((BN, 128), lambda i: (i, 0)),
            pl.BlockSpec((BN, 128), lambda i: (i, 0)),
            pl.BlockSpec((D, F), lambda i: (0, 0)),
            pl.BlockSpec((D, F), lambda i: (0, 0)),
            pl.BlockSpec((1, F), lambda i: (0, 0)),
        ],
        out_specs=pl.BlockSpec((BN, F), lambda i: (i, 0)),
        out_shape=jax.ShapeDtypeStruct((N, F), jnp.float32),
    )


def kernel(feats, edge_index, max_nodes, W0, b0, W1, b1):
    del max_nodes
    N, D = feats.shape
    E = edge_index.shape[1]
    F0 = W0.shape[1]
    F1 = W1.shape[1]
    src = edge_index[0]
    dst = edge_index[1]
    zrows = jnp.zeros((_C0, D), jnp.float32)
    ones_rows = jnp.ones((_K, D), jnp.float32)

    sc_agg = _make_sc_agg(N, D, E)

    deg = _make_sc_agg(N, D, E, False)(zrows, ones_rows, src, dst)
    agg0 = sc_agg(zrows, feats, src, dst)
    h1 = _make_dense(N, D, F0, False)(
        feats, agg0[0], agg0[1], deg[0], deg[1], W0[:D], W0[D:],
        b0.reshape(1, F0))
    agg1 = sc_agg(zrows, h1, src, dst)
    out = _make_dense(N, F0, F1, True)(
        h1, agg1[0], agg1[1], deg[0], deg[1], W1[:F0], W1[F0:],
        b1.reshape(1, F1))
    return out
